# baseline (speedup 1.0000x reference)
import functools

import jax
import jax.numpy as jnp
from jax import lax
from jax.experimental import pallas as pl
from jax.experimental.pallas import tpu as pltpu
from jax.experimental.pallas import tpu_sc as plsc

NUM_SEGMENTS = 256
E = 320000
D = 128

NC = 2
NS = 16
NW = NC * NS
ROWS_PER_TILE = E // NW
BLK = 128
GBLK = 256
NSLOT = 3
NGB = ROWS_PER_TILE // GBLK
NSUB = GBLK // BLK
NFULL = NSUB * NGB
TAIL = ROWS_PER_TILE - NFULL * BLK
IDROWS = NFULL + 2
SEGS_PER_TILE = NUM_SEGMENTS // NS
RUNROLL = 8
PROWS = 64

_mesh = plsc.VectorSubcoreMesh(core_axis_name="c", subcore_axis_name="s")


def _seg_sum_body(feat, ids2, out, fbuf, ibuf, iibuf, pacc, acc,
                  sem0, sem1, sem2, semi):
    c = lax.axis_index("c")
    s = lax.axis_index("s")
    sems = (sem0, sem1, sem2)
    w = s * NC + c
    base = w * ROWS_PER_TILE

    pltpu.async_copy(ids2.at[pl.ds(w * IDROWS, IDROWS)], ibuf, semi)

    zero16 = jnp.zeros((16,), jnp.float32)

    def zero_pacc(r, carry):
        for j in range(D // 16):
            pacc[r, pl.ds(j * 16, 16)] = zero16
        return carry

    lax.fori_loop(0, PROWS, zero_pacc, None)
    seg0 = s * SEGS_PER_TILE
    pltpu.sync_copy(
        pacc.at[pl.ds(0, SEGS_PER_TILE)],
        acc.at[pl.ds(seg0, SEGS_PER_TILE)])
    plsc.subcore_barrier()

    def start_gblock(g, b):
        for u in range(NSUB):
            pltpu.async_copy(
                feat.at[pl.ds(base + g * GBLK + u * BLK, BLK), :],
                fbuf.at[b, pl.ds(u * BLK, BLK)], sems[b])

    def wait_gblock(b):
        for u in range(NSUB):
            pltpu.make_async_copy(
                feat.at[pl.ds(0, BLK), :],
                fbuf.at[b, pl.ds(u * BLK, BLK)], sems[b]).wait()

    for b0 in range(NSLOT):
        start_gblock(b0, b0)

    pltpu.make_async_copy(ids2.at[pl.ds(0, IDROWS)], ibuf, semi).wait()

    firstseg = ibuf[0, pl.ds(0, 16)][0]
    iota16 = lax.iota(jnp.int32, 16)
    for j in range(PROWS // 16):
        iibuf[0, pl.ds(j * 16, 16)] = jnp.minimum(
            iota16 + (j * 16) + firstseg, NUM_SEGMENTS - 1)

    def process_sub(i, b, h):
        m = ibuf[i, pl.ds(0, 16)][0]
        mx = ibuf[i, pl.ds(BLK - 16, 16)][15]
        p = m - firstseg

        @pl.when((m == mx) & (p < PROWS))
        def _dense():
            def row_body(it, regs):
                new = regs
                for u in range(RUNROLL):
                    r = h * BLK + it * RUNROLL + u
                    new = tuple(
                        new[j] + fbuf[b, r, pl.ds(j * 16, 16)]
                        for j in range(D // 16))
                return new

            regs = lax.fori_loop(
                0, BLK // RUNROLL, row_body,
                tuple(jnp.zeros((16,), jnp.float32)
                      for _ in range(D // 16)))
            for j in range(D // 16):
                pacc[p, pl.ds(j * 16, 16)] = (
                    pacc[p, pl.ds(j * 16, 16)] + regs[j])

        @pl.when((m != mx) | (p >= PROWS))
        def _mixed():
            pltpu.sync_copy(
                fbuf.at[b, pl.ds(h * BLK, BLK)], acc.at[ibuf.at[i]],
                add=True)

    def loop_body(iv, carry):
        for b in range(NSLOT):
            g = NSLOT * iv + b
            wait_gblock(b)
            for h in range(NSUB):
                process_sub(NSUB * g + h, b, h)

            @pl.when(g + NSLOT < NGB)
            def _prefetch():
                start_gblock(g + NSLOT, b)
        return carry

    lax.fori_loop(0, NGB // NSLOT, loop_body, None)

    def zero_tail_row(r, carry):
        for j in range(D // 16):
            fbuf[0, r, pl.ds(j * 16, 16)] = zero16
        return carry

    lax.fori_loop(TAIL, BLK, zero_tail_row, None)
    pltpu.sync_copy(
        feat.at[pl.ds(base + NFULL * BLK, TAIL), :], fbuf.at[0, pl.ds(0, TAIL)])
    pltpu.sync_copy(
        fbuf.at[0, pl.ds(0, BLK)], acc.at[ibuf.at[NFULL]], add=True)

    pltpu.sync_copy(pacc, acc.at[iibuf.at[0]], add=True)

    plsc.subcore_barrier()
    pltpu.sync_copy(
        acc.at[pl.ds(seg0, SEGS_PER_TILE)],
        out.at[c, pl.ds(seg0, SEGS_PER_TILE), :])


_seg_sum = pl.kernel(
    _seg_sum_body,
    out_type=jax.ShapeDtypeStruct((NC, NUM_SEGMENTS, D), jnp.float32),
    mesh=_mesh,
    scratch_types=[
        pltpu.VMEM((NSLOT, GBLK, D), jnp.float32),
        pltpu.VMEM((IDROWS, BLK), jnp.int32),
        pltpu.VMEM((1, PROWS), jnp.int32),
        pltpu.VMEM((PROWS, D), jnp.float32),
        pltpu.VMEM_SHARED((NUM_SEGMENTS, D), jnp.float32),
        pltpu.SemaphoreType.DMA,
        pltpu.SemaphoreType.DMA,
        pltpu.SemaphoreType.DMA,
        pltpu.SemaphoreType.DMA,
    ],
)


def _combine_body(p_ref, o_ref):
    o_ref[...] = p_ref[0] + p_ref[1]


_combine = pl.pallas_call(
    _combine_body,
    out_shape=jax.ShapeDtypeStruct((NUM_SEGMENTS, D), jnp.float32),
)


def kernel(feat, segment_ids):
    ids2 = jnp.pad(
        segment_ids.reshape(NW, ROWS_PER_TILE),
        ((0, 0), (0, IDROWS * BLK - ROWS_PER_TILE)),
    ).reshape(NW * IDROWS, BLK)
    partials = _seg_sum(feat, ids2)
    return _combine(partials)
